# Initial kernel scaffold; baseline (speedup 1.0000x reference)
import jax
import jax.numpy as jnp
from jax.experimental import pallas as pl


def kernel(x, other_inputs_0, theta0, theta1, choice_indices):
    raise NotImplementedError("write your pallas kernel here")


# speedup vs baseline: 7.7212x; 7.7212x over previous
import jax
import jax.numpy as jnp
from jax.experimental import pallas as pl

_TEMP = 10.0
_ROW_BLOCK = 256


def _body(x_ref, o_ref, t0_ref, t1_ref, ci_ref, out_ref):
    x = x_ref[...]
    o = o_ref[...]
    t0 = t0_ref[...]
    t1 = t1_ref[...]
    ci = ci_ref[...]
    s0 = jax.nn.sigmoid(_TEMP * (jnp.abs(x) - t0))
    s1 = jax.nn.sigmoid(_TEMP * (jnp.abs(o) - t1))
    o0 = x * s0
    o1 = o * s1
    zeros = jnp.zeros((ci.shape[0], 2), dtype=ci.dtype)
    cip = jnp.concatenate([zeros, ci], axis=1)
    out_ref[...] = jnp.where(cip == 0, o0, o1)


def kernel(x, other_inputs_0, theta0, theta1, choice_indices):
    B, S, M = x.shape
    R = B * S
    xf = x.reshape(R, M)
    of = other_inputs_0.reshape(R, M)
    cif = choice_indices.reshape(R, M - 2)
    grid = R // _ROW_BLOCK
    out = pl.pallas_call(
        _body,
        grid=(grid,),
        in_specs=[
            pl.BlockSpec((_ROW_BLOCK, M), lambda i: (i, 0)),
            pl.BlockSpec((_ROW_BLOCK, M), lambda i: (i, 0)),
            pl.BlockSpec((1, M), lambda i: (0, 0)),
            pl.BlockSpec((1, M), lambda i: (0, 0)),
            pl.BlockSpec((_ROW_BLOCK, M - 2), lambda i: (i, 0)),
        ],
        out_specs=pl.BlockSpec((_ROW_BLOCK, M), lambda i: (i, 0)),
        out_shape=jax.ShapeDtypeStruct((R, M), jnp.float32),
    )(xf, of, theta0[None, :], theta1[None, :], cif)
    return out.reshape(B, S, M)
